# baseline (speedup 1.0000x reference)
import functools

import jax
import jax.numpy as jnp
from jax import lax
from jax.experimental import pallas as pl
from jax.experimental.pallas import tpu as pltpu
from jax.experimental.pallas import tpu_sc as plsc

_B = 16384
_C = 1000
_NC = 2
_NS = 16
_NW = _NC * _NS
_RPW = _B // _NW
_NG = _RPW // 16
_ROWPAD = 1008


def _sc_body(xflat, tgt, xt_out, neg_out,
             t_v, idxt_v, idx0_v, xt_v, x0_v, neg_v, row_v, sem):
    wid = lax.axis_index("s") * _NC + lax.axis_index("c")
    base = wid * _RPW
    lanes = lax.iota(jnp.int32, 16)

    pltpu.sync_copy(tgt.at[pl.ds(base, _RPW)], t_v)

    row_v[pl.ds(992, 16)] = jnp.zeros((16,), jnp.float32)

    def build(g, carry):
        t16 = t_v[pl.ds(g * 16, 16)]
        row16 = (base + g * 16) + lanes
        idxt_v[pl.ds(g * 16, 16)] = row16 * _C + t16
        idx0_v[pl.ds(g * 16, 16)] = row16 * _C
        return carry

    lax.fori_loop(0, _NG, build, 0)

    copies = []
    for c in range(4):
        sl = pl.ds(c * 128, 128)
        copies.append(pltpu.async_copy(xflat.at[idxt_v.at[sl]], xt_v.at[sl], sem))
        copies.append(pltpu.async_copy(xflat.at[idx0_v.at[sl]], x0_v.at[sl], sem))
    for cp in copies:
        cp.wait()

    def group(g, carry):
        gb = g * 16
        t16 = t_v[pl.ds(gb, 16)]
        x016 = x0_v[pl.ds(gb, 16)]
        e016 = jnp.exp(x016)
        neg_v[pl.ds(gb, 16)] = e016
        nzero = plsc.all_reduce_population_count(t16 == 0)[0]

        @pl.when(nzero > 0)
        def _():
            for l in range(16):
                t_l = t16[l]

                @pl.when(t_l == 0)
                def _zrow(l=l):
                    row = base + gb + l
                    pltpu.sync_copy(xflat.at[pl.ds(row * _C, _C)],
                                    row_v.at[pl.ds(0, _C)])

                    def sexp(j, acc):
                        return acc + jnp.exp(row_v[pl.ds(j * 16, 16)])

                    accv = lax.fori_loop(0, _ROWPAD // 16, sexp,
                                         jnp.zeros((16,), jnp.float32))
                    s = jnp.float32(-8.0)
                    for j in range(16):
                        s = s + accv[j]
                    cur = neg_v[pl.ds(gb, 16)]
                    neg_v[pl.ds(gb, 16)] = jnp.where(
                        lanes == l, s - e016[l], cur)

        return carry

    lax.fori_loop(0, _NG, group, 0)

    pltpu.sync_copy(xt_v, xt_out.at[pl.ds(base, _RPW)])
    pltpu.sync_copy(neg_v, neg_out.at[pl.ds(base, _RPW)])


def _fin_body(xt_ref, neg_ref, out_ref):
    xt = xt_ref[...]
    neg = neg_ref[...]
    out_ref[0, 0] = jnp.sum(jnp.log(neg + jnp.exp(xt)) - xt) / _B


@jax.jit
def kernel(inputs, targets):
    xflat = inputs.reshape(_B * _C)
    t32 = targets.astype(jnp.int32)

    mesh = plsc.VectorSubcoreMesh(core_axis_name="c", subcore_axis_name="s",
                                  num_cores=_NC, num_subcores=_NS)
    sc_fn = pl.kernel(
        _sc_body,
        out_type=[
            jax.ShapeDtypeStruct((_B,), jnp.float32),
            jax.ShapeDtypeStruct((_B,), jnp.float32),
        ],
        mesh=mesh,
        compiler_params=pltpu.CompilerParams(needs_layout_passes=False),
        scratch_types=[
            pltpu.VMEM((_RPW,), jnp.int32),
            pltpu.VMEM((_RPW,), jnp.int32),
            pltpu.VMEM((_RPW,), jnp.int32),
            pltpu.VMEM((_RPW,), jnp.float32),
            pltpu.VMEM((_RPW,), jnp.float32),
            pltpu.VMEM((_RPW,), jnp.float32),
            pltpu.VMEM((_ROWPAD,), jnp.float32),
            pltpu.SemaphoreType.DMA,
        ],
    )
    xt, neg = sc_fn(xflat, t32)

    loss = pl.pallas_call(
        _fin_body,
        out_specs=pl.BlockSpec(memory_space=pltpu.SMEM),
        out_shape=jax.ShapeDtypeStruct((1, 1), jnp.float32),
    )(xt.reshape(128, 128), neg.reshape(128, 128))
    return (loss[0, 0]).astype(inputs.dtype)


# speedup vs baseline: 1.7208x; 1.7208x over previous
import functools

import jax
import jax.numpy as jnp
from jax import lax
from jax.experimental import pallas as pl
from jax.experimental.pallas import tpu as pltpu
from jax.experimental.pallas import tpu_sc as plsc

_B = 16384
_C = 1000
_NC = 2
_NS = 16
_NW = _NC * _NS
_RPW = _B // _NW
_NG = _RPW // 16


def _sc_body(x2d, tgt, xt_out, neg_out,
             t_v, big_v, x0s_v, xt_v, neg_v, row_v, sem):
    wid = lax.axis_index("s") * _NC + lax.axis_index("c")
    base = wid * _RPW
    lanes = lax.iota(jnp.int32, 16)
    zero16 = jnp.zeros((16,), jnp.int32)

    pltpu.sync_copy(tgt.at[pl.ds(base, _RPW)], t_v)

    pltpu.sync_copy(x2d.at[pl.ds(base, _RPW), pl.ds(0, 128)], big_v)

    def getx0(g, carry):
        gb = g * 16
        x0s_v[pl.ds(gb, 16)] = plsc.load_gather(big_v, [gb + lanes, zero16])
        return carry

    lax.fori_loop(0, _NG, getx0, 0)

    def fire(g, carry):
        gb = g * 16
        t16 = t_v[pl.ds(gb, 16)]
        a16 = (t16 // 128) * 128
        for l in range(16):
            a_l = pl.multiple_of(a16[l], 128)
            pltpu.async_copy(
                x2d.at[base + gb + l, pl.ds(a_l, 128)],
                big_v.at[gb + l], sem)
        return carry

    lax.fori_loop(0, _NG, fire, 0)

    def drain(g, carry):
        for l in range(16):
            pltpu.make_async_copy(
                x2d.at[0, pl.ds(0, 128)], big_v.at[g * 16 + l], sem).wait()
        return carry

    lax.fori_loop(0, _NG, drain, 0)

    def group(g, carry):
        gb = g * 16
        t16 = t_v[pl.ds(gb, 16)]
        kidx = gb + lanes
        xt16 = plsc.load_gather(big_v, [kidx, t16 % 128])
        x016 = x0s_v[pl.ds(gb, 16)]
        xt_v[pl.ds(gb, 16)] = xt16
        e016 = jnp.exp(x016)
        neg_v[pl.ds(gb, 16)] = e016
        nzero = plsc.all_reduce_population_count(t16 == 0)[0]

        @pl.when(nzero > 0)
        def _():
            for l in range(16):
                t_l = t16[l]

                @pl.when(t_l == 0)
                def _zrow(l=l):
                    row = base + gb + l
                    pltpu.sync_copy(x2d.at[row, pl.ds(0, 896)],
                                    row_v.at[pl.ds(0, 896)])
                    s7 = pl.multiple_of(896 + t_l * 0, 128)
                    pltpu.sync_copy(x2d.at[row, pl.ds(s7, 128)],
                                    row_v.at[pl.ds(896, 128)])

                    def sexp(j, acc):
                        return acc + jnp.exp(row_v[pl.ds(j * 16, 16)])

                    accv = lax.fori_loop(0, 62, sexp,
                                         jnp.zeros((16,), jnp.float32))
                    tailv = row_v[pl.ds(992, 16)]
                    accv = accv + jnp.where(lanes < 8, jnp.exp(tailv), 0.0)
                    s = jnp.float32(0.0)
                    for j in range(16):
                        s = s + accv[j]
                    cur = neg_v[pl.ds(gb, 16)]
                    neg_v[pl.ds(gb, 16)] = jnp.where(
                        lanes == l, s - e016[l], cur)

        return carry

    lax.fori_loop(0, _NG, group, 0)

    pltpu.sync_copy(xt_v, xt_out.at[pl.ds(base, _RPW)])
    pltpu.sync_copy(neg_v, neg_out.at[pl.ds(base, _RPW)])


def _fin_body(xt_ref, neg_ref, out_ref):
    xt = xt_ref[...]
    neg = neg_ref[...]
    out_ref[0, 0] = jnp.sum(jnp.log(neg + jnp.exp(xt)) - xt) / _B


@jax.jit
def kernel(inputs, targets):
    t32 = targets.astype(jnp.int32)

    mesh = plsc.VectorSubcoreMesh(core_axis_name="c", subcore_axis_name="s",
                                  num_cores=_NC, num_subcores=_NS)
    sc_fn = pl.kernel(
        _sc_body,
        out_type=[
            jax.ShapeDtypeStruct((_B,), jnp.float32),
            jax.ShapeDtypeStruct((_B,), jnp.float32),
        ],
        mesh=mesh,
        compiler_params=pltpu.CompilerParams(needs_layout_passes=False),
        scratch_types=[
            pltpu.VMEM((_RPW,), jnp.int32),
            pltpu.VMEM((_RPW, 128), jnp.float32),
            pltpu.VMEM((_RPW,), jnp.float32),
            pltpu.VMEM((_RPW,), jnp.float32),
            pltpu.VMEM((_RPW,), jnp.float32),
            pltpu.VMEM((1024,), jnp.float32),
            pltpu.SemaphoreType.DMA,
        ],
    )
    xt, neg = sc_fn(inputs, t32)

    loss = pl.pallas_call(
        _fin_body,
        out_specs=pl.BlockSpec(memory_space=pltpu.SMEM),
        out_shape=jax.ShapeDtypeStruct((1, 1), jnp.float32),
    )(xt.reshape(128, 128), neg.reshape(128, 128))
    return (loss[0, 0]).astype(inputs.dtype)
